# baseline (speedup 1.0000x reference)
import functools

import jax
import jax.numpy as jnp
from jax.experimental import pallas as pl
from jax.experimental.pallas import tpu as pltpu

_CAPACITY = 2.0
_TILE = 128
_NTP = 128
_F = 2


def _gelu_tanh(v):
    return 0.5 * v * (1.0 + jnp.tanh(jnp.sqrt(2.0 / jnp.pi) * (v + 0.044715 * v ** 3)))


def _router_kernel(x_ref, gw_ref, bias_ref, comb_ref, cum_ref, meta_ref,
                   stats_ref, *, k):
    x = x_ref[...]
    gw = gw_ref[...]
    bias = bias_ref[...]
    S = x.shape[0]
    E = gw.shape[0]
    logits = jax.lax.dot_general(gw, x, (((1,), (1,)), ((), ())),
                                 preferred_element_type=jnp.float32)
    scores = jax.nn.sigmoid(logits + bias)
    si = jax.lax.bitcast_convert_type(scores, jnp.int32)

    def _tstep(i, t):
        cand = t | (jnp.int32(1) << (30 - i))
        cnt = jnp.sum((si >= cand).astype(jnp.int32), dtype=jnp.int32)
        return jnp.where(cnt >= k, cand, t)

    t = jax.lax.fori_loop(0, 31, _tstep, jnp.int32(0))

    gt = si > t
    eq = si == t
    cg = jnp.sum(gt.astype(jnp.int32), dtype=jnp.int32)
    need = k - cg

    e_iota = jax.lax.broadcasted_iota(jnp.int32, (E, S), 0)
    s_iota = jax.lax.broadcasted_iota(jnp.int32, (E, S), 1)
    fidx = e_iota * S + s_iota

    def _mstep(_, lohi):
        lo, hi = lohi
        mid = (lo + hi) // 2
        cnt = jnp.sum((eq & (fidx <= mid)).astype(jnp.int32), dtype=jnp.int32)
        return jnp.where(cnt >= need, lo, mid + 1), jnp.where(cnt >= need, mid, hi)

    lo, _ = jax.lax.fori_loop(0, 14, _mstep, (jnp.int32(0), jnp.int32(E * S - 1)))

    sel = gt | (eq & (fidx <= lo))
    self32 = sel.astype(jnp.float32)
    comb_ref[...] = jnp.where(sel, scores, 0.0)

    si_col = jax.lax.broadcasted_iota(jnp.int32, (S, S), 0)
    sj_col = jax.lax.broadcasted_iota(jnp.int32, (S, S), 1)
    tri = (si_col <= sj_col).astype(jnp.float32)
    cum = jax.lax.dot_general(self32, tri, (((1,), (0,)), ((), ())),
                              preferred_element_type=jnp.float32)
    cum_ref[...] = cum

    counts = cum[:, S - 1:S]
    stats_ref[...] = (counts / float(k)) * jnp.ones((E, _NTP), jnp.float32)

    nt = jnp.floor((counts + float(_TILE - 1)) / float(_TILE))
    li = jax.lax.broadcasted_iota(jnp.int32, (E, E), 0)
    lj = jax.lax.broadcasted_iota(jnp.int32, (E, E), 1)
    l8 = (lj < li).astype(jnp.float32)
    st = jax.lax.dot_general(l8, nt, (((1,), (0,)), ((), ())),
                             preferred_element_type=jnp.float32)
    en = st + nt
    tl = jax.lax.broadcasted_iota(jnp.int32, (E, _NTP), 1).astype(jnp.float32)
    active = ((tl >= st) & (tl < en)).astype(jnp.float32)
    e_col = jax.lax.broadcasted_iota(jnp.int32, (E, _NTP), 0).astype(jnp.float32)
    te = jnp.sum(active * e_col, axis=0, keepdims=True)
    rb = jnp.sum(active * (tl - st), axis=0, keepdims=True) * float(_TILE)
    vld = jnp.sum(active, axis=0, keepdims=True)
    te = jnp.where(vld > 0, te, float(E - 1))
    rb = jnp.where(vld > 0, rb, -1.0)
    meta_ref[0:1, :] = te.astype(jnp.int32)
    meta_ref[1:2, :] = rb.astype(jnp.int32)
    meta_ref[2:8, :] = jnp.zeros((6, _NTP), jnp.int32)


def _router_call(x_flat, gate_weight, expert_bias, k):
    S, D = x_flat.shape
    E = gate_weight.shape[0]
    return pl.pallas_call(
        functools.partial(_router_kernel, k=k),
        out_shape=(
            jax.ShapeDtypeStruct((E, S), jnp.float32),
            jax.ShapeDtypeStruct((E, S), jnp.float32),
            jax.ShapeDtypeStruct((8, _NTP), jnp.int32),
            jax.ShapeDtypeStruct((E, _NTP), jnp.float32),
        ),
    )(x_flat, gate_weight, expert_bias)


def _experts_kernel(te_ref, rb_ref, x_ref, w1_ref, b1_ref, w2_ref, b2_ref,
                    comb_ref, cum_ref, out_ref, xg_ref, acc_ref):
    t = pl.program_id(0)
    f = pl.program_id(1)

    @pl.when((t == 0) & (f == 0))
    def _():
        out_ref[...] = jnp.zeros_like(out_ref)

    @pl.when(rb_ref[t] >= 0)
    def _():
        rbv = rb_ref[t].astype(jnp.float32)
        cum = cum_ref[0]
        comb = comb_ref[0]
        rows = jax.lax.broadcasted_iota(jnp.int32, (_TILE, 1), 0).astype(jnp.float32)
        P = ((cum == (rbv + 1.0 + rows)) & (comb > 0.0)).astype(jnp.float32)

        @pl.when(f == 0)
        def _():
            xg_ref[...] = jax.lax.dot_general(
                P, x_ref[...], (((1,), (0,)), ((), ())),
                preferred_element_type=jnp.float32)

        h = jax.lax.dot_general(xg_ref[...], w1_ref[0], (((1,), (1,)), ((), ())),
                                preferred_element_type=jnp.float32)
        h = _gelu_tanh(h + b1_ref[0])
        part = jax.lax.dot_general(h, w2_ref[0], (((1,), (1,)), ((), ())),
                                   preferred_element_type=jnp.float32)

        @pl.when(f == 0)
        def _():
            acc_ref[...] = part

        @pl.when(f > 0)
        def _():
            acc_ref[...] += part

        @pl.when(f == _F - 1)
        def _():
            g = jax.lax.dot_general(P, comb, (((1,), (1,)), ((), ())),
                                    preferred_element_type=jnp.float32)
            yg = g * (acc_ref[...] + b2_ref[0])
            out_ref[...] += jax.lax.dot_general(
                P, yg, (((0,), (0,)), ((), ())),
                preferred_element_type=jnp.float32)


def _experts_call(x_flat, W1, b1, W2, b2, comb, cum, te, rb, nt_grid):
    S, D = x_flat.shape
    E, DFF, _ = W1.shape
    FT = DFF // _F
    grid_spec = pltpu.PrefetchScalarGridSpec(
        num_scalar_prefetch=2,
        grid=(nt_grid, _F),
        in_specs=[
            pl.BlockSpec((S, D), lambda t, f, te, rb: (0, 0)),
            pl.BlockSpec((1, FT, D), lambda t, f, te, rb: (te[t], f, 0)),
            pl.BlockSpec((1, 1, FT), lambda t, f, te, rb: (te[t] * _F + f, 0, 0)),
            pl.BlockSpec((1, D, FT), lambda t, f, te, rb: (te[t], 0, f)),
            pl.BlockSpec((1, 1, D), lambda t, f, te, rb: (te[t], 0, 0)),
            pl.BlockSpec((1, 1, S), lambda t, f, te, rb: (te[t], 0, 0)),
            pl.BlockSpec((1, 1, S), lambda t, f, te, rb: (te[t], 0, 0)),
        ],
        out_specs=pl.BlockSpec((S, D), lambda t, f, te, rb: (0, 0)),
        scratch_shapes=[
            pltpu.VMEM((_TILE, D), jnp.float32),
            pltpu.VMEM((_TILE, D), jnp.float32),
        ],
    )
    return pl.pallas_call(
        _experts_kernel,
        grid_spec=grid_spec,
        out_shape=jax.ShapeDtypeStruct((S, D), jnp.float32),
    )(te, rb, x_flat, W1, b1.reshape(E * _F, 1, FT), W2,
      b2.reshape(E, 1, D), comb.reshape(E, 1, S), cum.reshape(E, 1, S))


def kernel(x, gate_weight, expert_bias, W1, b1, W2, b2):
    Bsz, seq, D = x.shape
    E = gate_weight.shape[0]
    x_flat = x.reshape(-1, D)
    S = x_flat.shape[0]
    k = int(S * _CAPACITY)
    nt_grid = k // _TILE + E

    comb, cum, meta, stats = _router_call(x_flat, gate_weight, expert_bias, k)
    te = meta[0, :nt_grid]
    rb = meta[1, :nt_grid]
    out = _experts_call(x_flat, W1, b1, W2, b2, comb, cum, te, rb, nt_grid)

    x_out = out.reshape(Bsz, seq, D)
    token_each_expert = stats[:, 0]
    ones_like_mean = jnp.ones((E,), dtype=x.dtype)
    return (x_out, token_each_expert, ones_like_mean)


# speedup vs baseline: 1.4689x; 1.4689x over previous
import functools

import jax
import jax.numpy as jnp
from jax.experimental import pallas as pl
from jax.experimental.pallas import tpu as pltpu

_CAPACITY = 2.0
_TILE = 128
_NTP = 128
_F = 2


def _gelu_tanh(v):
    return 0.5 * v * (1.0 + jnp.tanh(jnp.sqrt(2.0 / jnp.pi) * (v + 0.044715 * v ** 3)))


def _router_kernel(x_ref, gw_ref, bias_ref, comb_ref, cum_ref, meta_ref,
                   stats_ref, *, k):
    x = x_ref[...]
    gw = gw_ref[...]
    bias = bias_ref[...]
    S = x.shape[0]
    E = gw.shape[0]
    logits = jax.lax.dot_general(gw, x, (((1,), (1,)), ((), ())),
                                 preferred_element_type=jnp.float32)
    scores = jax.nn.sigmoid(logits + bias)
    si = jax.lax.bitcast_convert_type(scores, jnp.int32)

    def _tstep(i, t):
        cand = t | (jnp.int32(1) << (30 - i))
        cnt = jnp.sum((si >= cand).astype(jnp.int32), dtype=jnp.int32)
        return jnp.where(cnt >= k, cand, t)

    t = jax.lax.fori_loop(0, 31, _tstep, jnp.int32(0))

    gt = si > t
    eq = si == t
    cg = jnp.sum(gt.astype(jnp.int32), dtype=jnp.int32)
    need = k - cg

    e_iota = jax.lax.broadcasted_iota(jnp.int32, (E, S), 0)
    s_iota = jax.lax.broadcasted_iota(jnp.int32, (E, S), 1)
    fidx = e_iota * S + s_iota

    def _mstep(_, lohi):
        lo, hi = lohi
        mid = (lo + hi) // 2
        cnt = jnp.sum((eq & (fidx <= mid)).astype(jnp.int32), dtype=jnp.int32)
        return jnp.where(cnt >= need, lo, mid + 1), jnp.where(cnt >= need, mid, hi)

    lo, _ = jax.lax.fori_loop(0, 14, _mstep, (jnp.int32(0), jnp.int32(E * S - 1)))

    sel = gt | (eq & (fidx <= lo))
    self32 = sel.astype(jnp.float32)
    comb_ref[...] = jnp.where(sel, scores, 0.0)

    si_col = jax.lax.broadcasted_iota(jnp.int32, (S, S), 0)
    sj_col = jax.lax.broadcasted_iota(jnp.int32, (S, S), 1)
    tri = (si_col <= sj_col).astype(jnp.bfloat16)
    cum = jax.lax.dot_general(self32.astype(jnp.bfloat16), tri,
                              (((1,), (0,)), ((), ())),
                              preferred_element_type=jnp.float32)
    cum_ref[...] = cum

    counts = cum[:, S - 1:S]
    stats_ref[...] = (counts / float(k)) * jnp.ones((E, _NTP), jnp.float32)

    nt = jnp.floor((counts + float(_TILE - 1)) / float(_TILE))
    li = jax.lax.broadcasted_iota(jnp.int32, (E, E), 0)
    lj = jax.lax.broadcasted_iota(jnp.int32, (E, E), 1)
    l8 = (lj < li).astype(jnp.float32)
    st = jax.lax.dot_general(l8, nt, (((1,), (0,)), ((), ())),
                             preferred_element_type=jnp.float32)
    en = st + nt
    tl = jax.lax.broadcasted_iota(jnp.int32, (E, _NTP), 1).astype(jnp.float32)
    active = ((tl >= st) & (tl < en)).astype(jnp.float32)
    e_col = jax.lax.broadcasted_iota(jnp.int32, (E, _NTP), 0).astype(jnp.float32)
    te = jnp.sum(active * e_col, axis=0, keepdims=True)
    rb = jnp.sum(active * (tl - st), axis=0, keepdims=True) * float(_TILE)
    vld = jnp.sum(active, axis=0, keepdims=True)
    te = jnp.where(vld > 0, te, float(E - 1))
    rb = jnp.where(vld > 0, rb, -1.0)
    meta_ref[0:1, :] = te.astype(jnp.int32)
    meta_ref[1:2, :] = rb.astype(jnp.int32)
    meta_ref[2:8, :] = jnp.zeros((6, _NTP), jnp.int32)


def _router_call(x_flat, gate_weight, expert_bias, k):
    S, D = x_flat.shape
    E = gate_weight.shape[0]
    return pl.pallas_call(
        functools.partial(_router_kernel, k=k),
        out_shape=(
            jax.ShapeDtypeStruct((E, S), jnp.float32),
            jax.ShapeDtypeStruct((E, S), jnp.float32),
            jax.ShapeDtypeStruct((8, _NTP), jnp.int32),
            jax.ShapeDtypeStruct((E, _NTP), jnp.float32),
        ),
    )(x_flat, gate_weight, expert_bias)


def _experts_kernel(te_ref, rb_ref, x_ref, w1_ref, b1_ref, w2_ref, b2_ref,
                    comb_ref, cum_ref, out_ref):
    t = pl.program_id(0)

    @pl.when(t == 0)
    def _():
        out_ref[...] = jnp.zeros_like(out_ref)

    @pl.when(rb_ref[t] >= 0)
    def _():
        rbv = rb_ref[t].astype(jnp.float32)
        cum = cum_ref[0]
        comb = comb_ref[0]
        rows = jax.lax.broadcasted_iota(jnp.int32, (_TILE, 1), 0).astype(jnp.float32)
        P = ((cum == (rbv + 1.0 + rows)) & (comb > 0.0)).astype(jnp.float32)

        xg = jax.lax.dot_general(P, x_ref[...], (((1,), (0,)), ((), ())),
                                 preferred_element_type=jnp.float32)
        h = jax.lax.dot_general(xg, w1_ref[0], (((1,), (1,)), ((), ())),
                                preferred_element_type=jnp.float32)
        h = _gelu_tanh(h + b1_ref[0])
        y = jax.lax.dot_general(h, w2_ref[0], (((1,), (1,)), ((), ())),
                                preferred_element_type=jnp.float32)
        g = jax.lax.dot_general(P, comb, (((1,), (1,)), ((), ())),
                                preferred_element_type=jnp.float32)
        yg = g * (y + b2_ref[0])
        out_ref[...] += jax.lax.dot_general(
            P, yg, (((0,), (0,)), ((), ())),
            preferred_element_type=jnp.float32)


def _experts_call(x_flat, W1, b1, W2, b2, comb, cum, te, rb, nt_grid):
    S, D = x_flat.shape
    E, DFF, _ = W1.shape
    grid_spec = pltpu.PrefetchScalarGridSpec(
        num_scalar_prefetch=2,
        grid=(nt_grid,),
        in_specs=[
            pl.BlockSpec((S, D), lambda t, te, rb: (0, 0)),
            pl.BlockSpec((1, DFF, D), lambda t, te, rb: (te[t], 0, 0)),
            pl.BlockSpec((1, 1, DFF), lambda t, te, rb: (te[t], 0, 0)),
            pl.BlockSpec((1, D, DFF), lambda t, te, rb: (te[t], 0, 0)),
            pl.BlockSpec((1, 1, D), lambda t, te, rb: (te[t], 0, 0)),
            pl.BlockSpec((1, 1, S), lambda t, te, rb: (te[t], 0, 0)),
            pl.BlockSpec((1, 1, S), lambda t, te, rb: (te[t], 0, 0)),
        ],
        out_specs=pl.BlockSpec((S, D), lambda t, te, rb: (0, 0)),
    )
    return pl.pallas_call(
        _experts_kernel,
        grid_spec=grid_spec,
        out_shape=jax.ShapeDtypeStruct((S, D), jnp.float32),
        compiler_params=pltpu.CompilerParams(
            vmem_limit_bytes=100 * 1024 * 1024),
    )(te, rb, x_flat, W1, b1.reshape(E, 1, DFF), W2,
      b2.reshape(E, 1, D), comb.reshape(E, 1, S), cum.reshape(E, 1, S))


def kernel(x, gate_weight, expert_bias, W1, b1, W2, b2):
    Bsz, seq, D = x.shape
    E = gate_weight.shape[0]
    x_flat = x.reshape(-1, D)
    S = x_flat.shape[0]
    k = int(S * _CAPACITY)
    nt_grid = k // _TILE + E

    comb, cum, meta, stats = _router_call(x_flat, gate_weight, expert_bias, k)
    te = meta[0, :nt_grid]
    rb = meta[1, :nt_grid]
    out = _experts_call(x_flat, W1, b1, W2, b2, comb, cum, te, rb, nt_grid)

    x_out = out.reshape(Bsz, seq, D)
    token_each_expert = stats[:, 0]
    ones_like_mean = jnp.ones((E,), dtype=x.dtype)
    return (x_out, token_each_expert, ones_like_mean)
